# baseline (speedup 1.0000x reference)
import functools

import jax
import jax.numpy as jnp
from jax import lax
from jax.experimental import pallas as pl
from jax.experimental.pallas import tpu as pltpu
from jax.experimental.pallas import tpu_sc as plsc

B = 16384
E = 64
H = 64
_W = 2 * E
_RBLK = 2048
_NRB = 488
_TAIL0 = 999424
_NTAIL = 576
_NLINES = _TAIL0 // 2

_NC = 2
_NS = 16
_NW = _NC * _NS
_ROWS_PER_W = B // _NW
_CHUNK = 128
_NCHUNK = _ROWS_PER_W // _CHUNK

_TC_BLK = 1024


def _sc_gather_body(table, *args):
    n = len(args) // 2 - 1
    idxs, outs, (idx_v, rows_v, sem) = args[:n], args[n:2 * n], args[2 * n:]
    wid = lax.axis_index("s") * _NC + lax.axis_index("c")
    base = wid * _ROWS_PER_W
    for idx_hbm, out_hbm in zip(idxs, outs):
        pltpu.sync_copy(idx_hbm.at[wid], idx_v)
        handles = []
        for ci in range(_NCHUNK):
            handles.append(pltpu.async_copy(
                table.at[idx_v.at[ci]],
                rows_v.at[pl.ds(ci * _CHUNK, _CHUNK)],
                sem))
        for h in handles:
            h.wait()
        pltpu.sync_copy(rows_v, out_hbm.at[pl.ds(base, _ROWS_PER_W)])


@functools.lru_cache(maxsize=2)
def _get_sc_gather(n, name):
    return pl.kernel(
        _sc_gather_body,
        out_type=[jax.ShapeDtypeStruct((B, _W), jnp.float32)] * n,
        name=name,
        mesh=plsc.VectorSubcoreMesh(core_axis_name="c", subcore_axis_name="s"),
        scratch_types=[
            pltpu.VMEM((_NCHUNK, _CHUNK), jnp.int32),
            pltpu.VMEM((_ROWS_PER_W, _W), jnp.float32),
            pltpu.SemaphoreType.DMA,
        ],
    )


def _repack_body(x_ref, out_ref):
    x = x_ref[...]
    rows = jax.lax.broadcasted_iota(jnp.int32, (E, E), 0)
    cols = jax.lax.broadcasted_iota(jnp.int32, (E, E), 1)
    eye = jnp.where(rows == cols, 1.0, 0.0)
    tr = lambda h: jax.lax.dot_general(
        h, eye, (((0,), (0,)), ((), ())), preferred_element_type=jnp.float32)
    out_ref[...] = jnp.concatenate(
        [tr(x[:, :_RBLK // 2]), tr(x[:, _RBLK // 2:])], axis=1)


def _repack(tableT):
    return pl.pallas_call(
        _repack_body,
        grid=(_NRB,),
        in_specs=[pl.BlockSpec((E, _RBLK), lambda i: (0, i))],
        out_specs=pl.BlockSpec((_RBLK // 2, _W), lambda i: (i, 0)),
        out_shape=jax.ShapeDtypeStruct((_NLINES, _W), jnp.float32),
        compiler_params=pltpu.CompilerParams(
            dimension_semantics=("arbitrary",)),
    )(tableT)


def _tc_body(gl_ref, gr_ref, gu_ref, g0_ref, g1_ref, g2_ref,
             sl_ref, sr_ref, su_ref, s0_ref, s1_ref, s2_ref,
             tl_ref, tr_ref, tu_ref, t0_ref, t1_ref, t2_ref,
             ti_ref, tu2_ref,
             wn_ref, bn_ref, wu_ref, bu_ref, wc_ref, bc_ref,
             w1_ref, b1_ref, wo_ref, scal_ref, out_ref):
    relu = lambda x: jnp.maximum(x, 0.0)
    mm = lambda a, w: jax.lax.dot_general(
        a, w, (((1,), (0,)), ((), ())), preferred_element_type=jnp.float32)

    def pick(ref, side_ref, toff_ref, ttab_ref):
        x = ref[...]
        base = jnp.where(side_ref[...] > 0.5, x[:, E:], x[:, :E])
        toff = toff_ref[...]
        iota = jax.lax.broadcasted_iota(jnp.int32, (x.shape[0], _NTAIL), 1)
        onehot = jnp.where(iota == toff, 1.0, 0.0)
        tail = mm(onehot, ttab_ref[...])
        return jnp.where(toff >= 0, tail, base)

    d0 = scal_ref[0]
    d1 = scal_ref[1]
    d2 = scal_ref[2]
    bo = scal_ref[3]
    wn = wn_ref[...]
    bn = bn_ref[...]
    wc = wc_ref[...]
    bc = bc_ref[...]
    left = mm(relu(pick(gl_ref, sl_ref, tl_ref, ti_ref)), wn) + bn
    right = mm(relu(pick(gr_ref, sr_ref, tr_ref, ti_ref)), wn) + bn
    user = mm(relu(pick(gu_ref, su_ref, tu_ref, tu2_ref)), wu_ref[...]) + bu_ref[...]
    casc = mm(relu(pick(g2_ref, s2_ref, t2_ref, ti_ref) + d0), wc) + bc
    casc = mm(relu(casc + pick(g1_ref, s1_ref, t1_ref, ti_ref) + d1), wc) + bc
    casc = mm(relu(casc + pick(g0_ref, s0_ref, t0_ref, ti_ref) + d2), wc) + bc
    common = user + casc
    w1 = w1_ref[...]
    b1 = b1_ref[...]
    wo = wo_ref[...]

    def half(x):
        h = relu(mm(relu(x), w1) + b1)
        z = jnp.sum(h * wo, axis=1, keepdims=True) + bo
        return jax.nn.sigmoid(z)

    out_ref[...] = half(left + common) - half(right + common)


def _tc_forward(gs, sides, toffs, ti, tu2,
                wn, bn, wu, bu, wc, bc, w1, b1, wo, scal):
    row_spec = pl.BlockSpec((_TC_BLK, _W), lambda i: (i, 0))
    side_spec = pl.BlockSpec((_TC_BLK, 1), lambda i: (i, 0))
    tail_spec = pl.BlockSpec((_NTAIL, E), lambda i: (0, 0))
    w_spec = pl.BlockSpec((E, H), lambda i: (0, 0))
    b_spec = pl.BlockSpec((1, H), lambda i: (0, 0))
    return pl.pallas_call(
        _tc_body,
        grid=(B // _TC_BLK,),
        in_specs=[row_spec] * 6 + [side_spec] * 6 + [side_spec] * 6 +
                 [tail_spec] * 2 +
                 [w_spec, b_spec, w_spec, b_spec, w_spec, b_spec,
                  w_spec, b_spec, b_spec,
                  pl.BlockSpec(memory_space=pltpu.SMEM)],
        out_specs=pl.BlockSpec((_TC_BLK, 1), lambda i: (i, 0)),
        out_shape=jax.ShapeDtypeStruct((B, 1), jnp.float32),
        compiler_params=pltpu.CompilerParams(
            dimension_semantics=("parallel",)),
    )(*gs, *sides, *toffs, ti, tu2, wn, bn, wu, bu, wc, bc, w1, b1, wo, scal)


def _prep_idx(x):
    x = x.astype(jnp.int32)
    line = ((x >> 11) << 10) | (x & 1023)
    line = jnp.minimum(line, _NLINES - 1).reshape(_NW, _NCHUNK, _CHUNK)
    side = ((x >> 10) & 1).astype(jnp.float32).reshape(B, 1)
    toff = (x - _TAIL0).reshape(B, 1)
    return line, side, toff


def kernel(users, left_items, right_items, prev_item_0, prev_item_1,
           prev_item_2, item_emb, user_emb, W_user, b_user, W_next, b_next,
           W_casc, b_casc, d0, d1, d2, W1, b1, Wo, bo):
    li, sl, tl = _prep_idx(left_items)
    ri, sr, tr = _prep_idx(right_items)
    us, su, tu = _prep_idx(users)
    p0, s0, t0 = _prep_idx(prev_item_0)
    p1, s1, t1 = _prep_idx(prev_item_1)
    p2, s2, t2 = _prep_idx(prev_item_2)
    item2 = _repack(item_emb.T)
    user2 = _repack(user_emb.T)
    ti = jax.lax.slice(item_emb, (_TAIL0, 0), (_TAIL0 + _NTAIL, E))
    tu2 = jax.lax.slice(user_emb, (_TAIL0, 0), (_TAIL0 + _NTAIL, E))
    gl, gr, g0, g1, g2 = _get_sc_gather(5, "sc_item_gather")(
        item2, li, ri, p0, p1, p2)
    (gu,) = _get_sc_gather(1, "sc_user_gather")(user2, us)
    gs = (gl, gr, gu, g0, g1, g2)
    scal = jnp.concatenate([d0, d1, d2, bo]).astype(jnp.float32)
    return _tc_forward(
        gs, (sl, sr, su, s0, s1, s2), (tl, tr, tu, t0, t1, t2), ti, tu2,
        W_next, b_next.reshape(1, H),
        W_user, b_user.reshape(1, H),
        W_casc, b_casc.reshape(1, H),
        W1, b1.reshape(1, H),
        Wo.reshape(1, H), scal)


# speedup vs baseline: 1.3198x; 1.3198x over previous
import functools

import jax
import jax.numpy as jnp
from jax import lax
from jax.experimental import pallas as pl
from jax.experimental.pallas import tpu as pltpu
from jax.experimental.pallas import tpu_sc as plsc

B = 16384
E = 64
H = 64
_W = 2 * E
_RBLK = 2048
_NRB = 245
_POFF = _RBLK * _NRB
_TAIL0 = 999424
_NTAIL = 576

_NC = 2
_NS = 16
_NW = _NC * _NS
_ROWS_PER_W = B // _NW
_CHUNK = 128
_NCHUNK = _ROWS_PER_W // _CHUNK

_TC_BLK = 1024


def _sc_gather_body(table, *args):
    n = len(args) // 2 - 1
    idxs, outs, (idx_v, rows_v, sem) = args[:n], args[n:2 * n], args[2 * n:]
    wid = lax.axis_index("s") * _NC + lax.axis_index("c")
    base = wid * _ROWS_PER_W
    for idx_hbm, out_hbm in zip(idxs, outs):
        pltpu.sync_copy(idx_hbm.at[wid], idx_v)
        handles = []
        for ci in range(_NCHUNK):
            handles.append(pltpu.async_copy(
                table.at[idx_v.at[ci]],
                rows_v.at[pl.ds(ci * _CHUNK, _CHUNK)],
                sem))
        for h in handles:
            h.wait()
        pltpu.sync_copy(rows_v, out_hbm.at[pl.ds(base, _ROWS_PER_W)])


@functools.lru_cache(maxsize=2)
def _get_sc_gather(n, name):
    return pl.kernel(
        _sc_gather_body,
        out_type=[jax.ShapeDtypeStruct((B, _W), jnp.float32)] * n,
        name=name,
        mesh=plsc.VectorSubcoreMesh(core_axis_name="c", subcore_axis_name="s"),
        scratch_types=[
            pltpu.VMEM((_NCHUNK, _CHUNK), jnp.int32),
            pltpu.VMEM((_ROWS_PER_W, _W), jnp.float32),
            pltpu.SemaphoreType.DMA,
        ],
    )


def _repack_body(lo_ref, hi_ref, out_ref):
    out_ref[...] = jnp.concatenate([lo_ref[...].T, hi_ref[...].T], axis=1)


def _repack(tableT):
    return pl.pallas_call(
        _repack_body,
        grid=(_NRB,),
        in_specs=[pl.BlockSpec((E, _RBLK), lambda i: (0, i)),
                  pl.BlockSpec((E, _RBLK),
                               lambda i: (0, jnp.where(i < 243, i + 245, 0)))],
        out_specs=pl.BlockSpec((_RBLK, _W), lambda i: (i, 0)),
        out_shape=jax.ShapeDtypeStruct((_POFF, _W), jnp.float32),
        compiler_params=pltpu.CompilerParams(
            dimension_semantics=("arbitrary",)),
    )(tableT, tableT)


def _tc_body(gl_ref, gr_ref, gu_ref, g0_ref, g1_ref, g2_ref,
             sl_ref, sr_ref, su_ref, s0_ref, s1_ref, s2_ref,
             tl_ref, tr_ref, tu_ref, t0_ref, t1_ref, t2_ref,
             ti_ref, tu2_ref,
             wn_ref, bn_ref, wu_ref, bu_ref, wc_ref, bc_ref,
             w1_ref, b1_ref, wo_ref, scal_ref, out_ref):
    relu = lambda x: jnp.maximum(x, 0.0)
    mm = lambda a, w: jax.lax.dot_general(
        a, w, (((1,), (0,)), ((), ())), preferred_element_type=jnp.float32)

    def pick(ref, side_ref, toff_ref, ttab_ref):
        x = ref[...]
        base = jnp.where(side_ref[...] > 0.5, x[:, E:], x[:, :E])
        toff = toff_ref[...]
        iota = jax.lax.broadcasted_iota(jnp.int32, (x.shape[0], _NTAIL), 1)
        onehot = jnp.where(iota == toff, 1.0, 0.0)
        tail = mm(onehot, ttab_ref[...])
        return jnp.where(toff >= 0, tail, base)

    d0 = scal_ref[0]
    d1 = scal_ref[1]
    d2 = scal_ref[2]
    bo = scal_ref[3]
    wn = wn_ref[...]
    bn = bn_ref[...]
    wc = wc_ref[...]
    bc = bc_ref[...]
    left = mm(relu(pick(gl_ref, sl_ref, tl_ref, ti_ref)), wn) + bn
    right = mm(relu(pick(gr_ref, sr_ref, tr_ref, ti_ref)), wn) + bn
    user = mm(relu(pick(gu_ref, su_ref, tu_ref, tu2_ref)), wu_ref[...]) + bu_ref[...]
    casc = mm(relu(pick(g2_ref, s2_ref, t2_ref, ti_ref) + d0), wc) + bc
    casc = mm(relu(casc + pick(g1_ref, s1_ref, t1_ref, ti_ref) + d1), wc) + bc
    casc = mm(relu(casc + pick(g0_ref, s0_ref, t0_ref, ti_ref) + d2), wc) + bc
    common = user + casc
    w1 = w1_ref[...]
    b1 = b1_ref[...]
    wo = wo_ref[...]

    def half(x):
        h = relu(mm(relu(x), w1) + b1)
        z = jnp.sum(h * wo, axis=1, keepdims=True) + bo
        return jax.nn.sigmoid(z)

    out_ref[...] = half(left + common) - half(right + common)


def _tc_forward(gs, sides, toffs, ti, tu2,
                wn, bn, wu, bu, wc, bc, w1, b1, wo, scal):
    row_spec = pl.BlockSpec((_TC_BLK, _W), lambda i: (i, 0))
    side_spec = pl.BlockSpec((_TC_BLK, 1), lambda i: (i, 0))
    tail_spec = pl.BlockSpec((_NTAIL, E), lambda i: (0, 0))
    w_spec = pl.BlockSpec((E, H), lambda i: (0, 0))
    b_spec = pl.BlockSpec((1, H), lambda i: (0, 0))
    return pl.pallas_call(
        _tc_body,
        grid=(B // _TC_BLK,),
        in_specs=[row_spec] * 6 + [side_spec] * 6 + [side_spec] * 6 +
                 [tail_spec] * 2 +
                 [w_spec, b_spec, w_spec, b_spec, w_spec, b_spec,
                  w_spec, b_spec, b_spec,
                  pl.BlockSpec(memory_space=pltpu.SMEM)],
        out_specs=pl.BlockSpec((_TC_BLK, 1), lambda i: (i, 0)),
        out_shape=jax.ShapeDtypeStruct((B, 1), jnp.float32),
        compiler_params=pltpu.CompilerParams(
            dimension_semantics=("parallel",)),
    )(*gs, *sides, *toffs, ti, tu2, wn, bn, wu, bu, wc, bc, w1, b1, wo, scal)


def _prep_idx(x):
    x = x.astype(jnp.int32)
    line = jnp.where(x < _POFF, x, x - _POFF).reshape(_NW, _NCHUNK, _CHUNK)
    side = (x >= _POFF).astype(jnp.float32).reshape(B, 1)
    toff = (x - _TAIL0).reshape(B, 1)
    return line, side, toff


def kernel(users, left_items, right_items, prev_item_0, prev_item_1,
           prev_item_2, item_emb, user_emb, W_user, b_user, W_next, b_next,
           W_casc, b_casc, d0, d1, d2, W1, b1, Wo, bo):
    li, sl, tl = _prep_idx(left_items)
    ri, sr, tr = _prep_idx(right_items)
    us, su, tu = _prep_idx(users)
    p0, s0, t0 = _prep_idx(prev_item_0)
    p1, s1, t1 = _prep_idx(prev_item_1)
    p2, s2, t2 = _prep_idx(prev_item_2)
    item2 = _repack(item_emb.T)
    user2 = _repack(user_emb.T)
    ti = jax.lax.slice(item_emb, (_TAIL0, 0), (_TAIL0 + _NTAIL, E))
    tu2 = jax.lax.slice(user_emb, (_TAIL0, 0), (_TAIL0 + _NTAIL, E))
    gl, gr, g0, g1, g2 = _get_sc_gather(5, "sc_item_gather")(
        item2, li, ri, p0, p1, p2)
    (gu,) = _get_sc_gather(1, "sc_user_gather")(user2, us)
    gs = (gl, gr, gu, g0, g1, g2)
    scal = jnp.concatenate([d0, d1, d2, bo]).astype(jnp.float32)
    return _tc_forward(
        gs, (sl, sr, su, s0, s1, s2), (tl, tr, tu, t0, t1, t2), ti, tu2,
        W_next, b_next.reshape(1, H),
        W_user, b_user.reshape(1, H),
        W_casc, b_casc.reshape(1, H),
        W1, b1.reshape(1, H),
        Wo.reshape(1, H), scal)
